# baseline (speedup 1.0000x reference)
import functools

import jax
import jax.numpy as jnp
from jax import lax
from jax.experimental import pallas as pl
from jax.experimental.pallas import tpu as pltpu
from jax.experimental.pallas import tpu_sc as plsc

QK_DIM = 32
TOPK = 16
SCALE = QK_DIM ** (-0.5)
MM_BLOCK = 16
ROW_BLOCK = 64
NUM_WORKERS = 32


def _logits_body(q_ref, k_ref, x_ref):
    x_ref[...] = lax.dot_general(
        q_ref[...] * SCALE, k_ref[...],
        dimension_numbers=(((2,), (2,)), ((0,), (0,))),
        preferred_element_type=jnp.float32,
    )


def _tc_logits(query, key):
    n, p, d = query.shape
    b = MM_BLOCK
    return pl.pallas_call(
        _logits_body,
        grid=(n // b,),
        in_specs=[
            pl.BlockSpec((b, p, d), lambda i: (i, 0, 0)),
            pl.BlockSpec((b, p, d), lambda i: (i, 0, 0)),
        ],
        out_specs=pl.BlockSpec((b, p, p), lambda i: (i, 0, 0)),
        out_shape=jax.ShapeDtypeStruct((n, p, p), jnp.float32),
    )(query, key)


def _sc_topk_body(x_hbm, w_hbm, i_hbm, xbuf, wbuf, ibuf):
    nrows, p = x_hbm.shape
    n = w_hbm.shape[0]
    rows_per_w = nrows // NUM_WORKERS
    nblk = rows_per_w // ROW_BLOCK
    blk_per_batch = p // ROW_BLOCK
    nchunk = p // 16
    wid = lax.axis_index("s") * 2 + lax.axis_index("c")
    lane = lax.broadcasted_iota(jnp.int32, (16,), 0)

    def blk_body(b, _):
        base = wid * rows_per_w + b * ROW_BLOCK
        nb = base // p
        r0 = (b % blk_per_batch) * ROW_BLOCK
        pltpu.sync_copy(x_hbm.at[pl.ds(base, ROW_BLOCK)], xbuf)

        @plsc.parallel_loop(0, ROW_BLOCK, unroll=8)
        def row_body(r):
            level = []
            for c in range(nchunk):
                v, i = plsc.sort_key_val(
                    xbuf[r, pl.ds(c * 16, 16)], lane + c * 16,
                    descending=(c % 2 == 0))
                level.append((v, i))
            while len(level) > 1:
                nxt = []
                for j in range(0, len(level), 2):
                    av, ai = level[j]
                    bv, bi = level[j + 1]
                    ge = av >= bv
                    nv = jnp.where(ge, av, bv)
                    ni = jnp.where(ge, ai, bi)
                    desc = True if len(level) == 2 else (j // 2) % 2 == 0
                    nxt.append(plsc.sort_key_val(nv, ni, descending=desc))
                level = nxt
            ov, oi = level[0]
            m = lax.reduce_max(ov, axes=(0,))
            e = jnp.exp(ov - m)
            s = lax.reduce_sum(e, axes=(0,))
            wbuf[r] = e / s
            ibuf[r] = oi

        pltpu.sync_copy(wbuf, w_hbm.at[pl.ds(base, ROW_BLOCK)])
        pltpu.sync_copy(ibuf, i_hbm.at[pl.ds(base, ROW_BLOCK)])
        return 0

    lax.fori_loop(0, nblk, blk_body, 0)


def _sc_topk(x, n):
    nrows, p = x.shape
    mesh = plsc.VectorSubcoreMesh(
        core_axis_name="c", subcore_axis_name="s",
        num_cores=2, num_subcores=16)
    f = functools.partial(
        pl.kernel,
        mesh=mesh,
        out_type=[
            jax.ShapeDtypeStruct((nrows, TOPK), jnp.float32),
            jax.ShapeDtypeStruct((nrows, TOPK), jnp.int32),
        ],
        scratch_types=[
            pltpu.VMEM((ROW_BLOCK, p), jnp.float32),
            pltpu.VMEM((ROW_BLOCK, TOPK), jnp.float32),
            pltpu.VMEM((ROW_BLOCK, TOPK), jnp.int32),
        ],
        compiler_params=pltpu.CompilerParams(needs_layout_passes=False),
    )(_sc_topk_body)
    return f(x)


@jax.jit
def kernel(query, key):
    n, p, d = query.shape
    x = _tc_logits(query, key)
    w, i = _sc_topk(x.reshape(n * p, p), n)
    return w.reshape(n, p, TOPK), i.reshape(n, p, TOPK)


# speedup vs baseline: 1.3111x; 1.3111x over previous
import functools

import jax
import jax.numpy as jnp
from jax import lax
from jax.experimental import pallas as pl
from jax.experimental.pallas import tpu as pltpu
from jax.experimental.pallas import tpu_sc as plsc

QK_DIM = 32
TOPK = 16
SCALE = QK_DIM ** (-0.5)
MM_BLOCK = 16
ROW_BLOCK = 64
NUM_WORKERS = 32


def _logits_body(q_ref, k_ref, x_ref):
    x_ref[...] = lax.dot_general(
        q_ref[...] * SCALE, k_ref[...],
        dimension_numbers=(((2,), (2,)), ((0,), (0,))),
        preferred_element_type=jnp.float32,
    )


def _tc_logits(query, key):
    n, p, d = query.shape
    b = MM_BLOCK
    return pl.pallas_call(
        _logits_body,
        grid=(n // b,),
        in_specs=[
            pl.BlockSpec((b, p, d), lambda i: (i, 0, 0)),
            pl.BlockSpec((b, p, d), lambda i: (i, 0, 0)),
        ],
        out_specs=pl.BlockSpec((b, p, p), lambda i: (i, 0, 0)),
        out_shape=jax.ShapeDtypeStruct((n, p, p), jnp.float32),
    )(query, key)


def _sc_topk_body(x_hbm, w_hbm, i_hbm, xbuf, wbuf, ibuf):
    nrows, p = x_hbm.shape
    n = w_hbm.shape[0]
    rows_per_w = nrows // NUM_WORKERS
    nblk = rows_per_w // ROW_BLOCK
    blk_per_batch = p // ROW_BLOCK
    nchunk = p // 16
    wid = lax.axis_index("s") * 2 + lax.axis_index("c")
    lane = lax.broadcasted_iota(jnp.int32, (16,), 0)

    def blk_body(b, _):
        base = wid * rows_per_w + b * ROW_BLOCK
        nb = base // p
        r0 = (b % blk_per_batch) * ROW_BLOCK
        pltpu.sync_copy(x_hbm.at[pl.ds(base, ROW_BLOCK)], xbuf)

        @plsc.parallel_loop(0, ROW_BLOCK, unroll=2)
        def row_body(r):
            level = []
            for c in range(nchunk):
                v, i = plsc.sort_key_val(
                    xbuf[r, pl.ds(c * 16, 16)], lane + c * 16,
                    descending=(c % 2 == 0))
                level.append((v, i))
            while len(level) > 1:
                nxt = []
                for j in range(0, len(level), 2):
                    av, ai = level[j]
                    bv, bi = level[j + 1]
                    ge = av >= bv
                    nv = jnp.where(ge, av, bv)
                    ni = jnp.where(ge, ai, bi)
                    desc = True if len(level) == 2 else (j // 2) % 2 == 0
                    nxt.append(plsc.sort_key_val(nv, ni, descending=desc))
                level = nxt
            ov, oi = level[0]
            m = lax.reduce_max(ov, axes=(0,))
            e = jnp.exp(ov - m)
            s = lax.reduce_sum(e, axes=(0,))
            wbuf[r] = e / s
            ibuf[r] = oi

        pltpu.sync_copy(wbuf, w_hbm.at[pl.ds(base, ROW_BLOCK)])
        pltpu.sync_copy(ibuf, i_hbm.at[pl.ds(base, ROW_BLOCK)])
        return 0

    lax.fori_loop(0, nblk, blk_body, 0)


def _sc_topk(x, n):
    nrows, p = x.shape
    mesh = plsc.VectorSubcoreMesh(
        core_axis_name="c", subcore_axis_name="s",
        num_cores=2, num_subcores=16)
    f = functools.partial(
        pl.kernel,
        mesh=mesh,
        out_type=[
            jax.ShapeDtypeStruct((nrows, TOPK), jnp.float32),
            jax.ShapeDtypeStruct((nrows, TOPK), jnp.int32),
        ],
        scratch_types=[
            pltpu.VMEM((ROW_BLOCK, p), jnp.float32),
            pltpu.VMEM((ROW_BLOCK, TOPK), jnp.float32),
            pltpu.VMEM((ROW_BLOCK, TOPK), jnp.int32),
        ],
        compiler_params=pltpu.CompilerParams(needs_layout_passes=False),
    )(_sc_topk_body)
    return f(x)


@jax.jit
def kernel(query, key):
    n, p, d = query.shape
    x = _tc_logits(query, key)
    w, i = _sc_topk(x.reshape(n * p, p), n)
    return w.reshape(n, p, TOPK), i.reshape(n, p, TOPK)


# speedup vs baseline: 1.6741x; 1.2768x over previous
import functools

import jax
import jax.numpy as jnp
from jax import lax
from jax.experimental import pallas as pl
from jax.experimental.pallas import tpu as pltpu
from jax.experimental.pallas import tpu_sc as plsc

QK_DIM = 32
TOPK = 16
SCALE = QK_DIM ** (-0.5)
MM_BLOCK = 16
ROW_BLOCK = 64
NUM_WORKERS = 32


def _logits_body(q_ref, k_ref, x_ref):
    x_ref[...] = lax.dot_general(
        q_ref[...] * SCALE, k_ref[...],
        dimension_numbers=(((2,), (2,)), ((0,), (0,))),
        preferred_element_type=jnp.float32,
    )


def _tc_logits(query, key):
    n, p, d = query.shape
    b = MM_BLOCK
    return pl.pallas_call(
        _logits_body,
        grid=(n // b,),
        in_specs=[
            pl.BlockSpec((b, p, d), lambda i: (i, 0, 0)),
            pl.BlockSpec((b, p, d), lambda i: (i, 0, 0)),
        ],
        out_specs=pl.BlockSpec((b, p, p), lambda i: (i, 0, 0)),
        out_shape=jax.ShapeDtypeStruct((n, p, p), jnp.float32),
    )(query, key)


def _sc_topk_body(x_hbm, w_hbm, i_hbm, xbuf0, xbuf1, wbuf, ibuf, sem0, sem1):
    nrows, p = x_hbm.shape
    rows_per_w = nrows // NUM_WORKERS
    nblk = rows_per_w // ROW_BLOCK
    nchunk = p // 16
    wid = lax.axis_index("s") * 2 + lax.axis_index("c")
    wbase = wid * rows_per_w
    lane = lax.broadcasted_iota(jnp.int32, (16,), 0)

    def compute_block(xbuf):
        @plsc.parallel_loop(0, ROW_BLOCK, unroll=2)
        def row_body(r):
            level = []
            for c in range(nchunk):
                v, i = plsc.sort_key_val(
                    xbuf[r, pl.ds(c * 16, 16)], lane + c * 16,
                    descending=(c % 2 == 0))
                level.append((v, i))
            while len(level) > 1:
                nxt = []
                for j in range(0, len(level), 2):
                    av, ai = level[j]
                    bv, bi = level[j + 1]
                    ge = av >= bv
                    nv = jnp.where(ge, av, bv)
                    ni = jnp.where(ge, ai, bi)
                    desc = True if len(level) == 2 else (j // 2) % 2 == 0
                    nxt.append(plsc.sort_key_val(nv, ni, descending=desc))
                level = nxt
            ov, oi = level[0]
            m = lax.reduce_max(ov, axes=(0,))
            e = jnp.exp(ov - m)
            s = lax.reduce_sum(e, axes=(0,))
            wbuf[r] = e / s
            ibuf[r] = oi

    pltpu.async_copy(x_hbm.at[pl.ds(wbase, ROW_BLOCK)], xbuf0, sem0)
    pltpu.async_copy(x_hbm.at[pl.ds(wbase + ROW_BLOCK, ROW_BLOCK)], xbuf1, sem1)

    def pair_body(b2, _):
        for ph, (xb, sem) in enumerate(((xbuf0, sem0), (xbuf1, sem1))):
            b = 2 * b2 + ph
            base = wbase + b * ROW_BLOCK
            pltpu.make_async_copy(
                x_hbm.at[pl.ds(base, ROW_BLOCK)], xb, sem).wait()
            compute_block(xb)

            @pl.when(b + 2 < nblk)
            def _():
                pltpu.async_copy(
                    x_hbm.at[pl.ds(base + 2 * ROW_BLOCK, ROW_BLOCK)], xb, sem)

            pltpu.sync_copy(wbuf, w_hbm.at[pl.ds(base, ROW_BLOCK)])
            pltpu.sync_copy(ibuf, i_hbm.at[pl.ds(base, ROW_BLOCK)])
        return 0

    lax.fori_loop(0, nblk // 2, pair_body, 0)


def _sc_topk(x, n):
    nrows, p = x.shape
    mesh = plsc.VectorSubcoreMesh(
        core_axis_name="c", subcore_axis_name="s",
        num_cores=2, num_subcores=16)
    f = functools.partial(
        pl.kernel,
        mesh=mesh,
        out_type=[
            jax.ShapeDtypeStruct((nrows, TOPK), jnp.float32),
            jax.ShapeDtypeStruct((nrows, TOPK), jnp.int32),
        ],
        scratch_types=[
            pltpu.VMEM((ROW_BLOCK, p), jnp.float32),
            pltpu.VMEM((ROW_BLOCK, p), jnp.float32),
            pltpu.VMEM((ROW_BLOCK, TOPK), jnp.float32),
            pltpu.VMEM((ROW_BLOCK, TOPK), jnp.int32),
            pltpu.SemaphoreType.DMA,
            pltpu.SemaphoreType.DMA,
        ],
        compiler_params=pltpu.CompilerParams(needs_layout_passes=False),
    )(_sc_topk_body)
    return f(x)


@jax.jit
def kernel(query, key):
    n, p, d = query.shape
    x = _tc_logits(query, key)
    w, i = _sc_topk(x.reshape(n * p, p), n)
    return w.reshape(n, p, TOPK), i.reshape(n, p, TOPK)


# speedup vs baseline: 1.7092x; 1.0210x over previous
import functools

import jax
import jax.numpy as jnp
from jax import lax
from jax.experimental import pallas as pl
from jax.experimental.pallas import tpu as pltpu
from jax.experimental.pallas import tpu_sc as plsc

QK_DIM = 32
TOPK = 16
SCALE = QK_DIM ** (-0.5)
MM_BLOCK = 16
ROW_BLOCK = 128
NUM_WORKERS = 32


def _logits_body(q_ref, k_ref, x_ref):
    x_ref[...] = lax.dot_general(
        q_ref[...] * SCALE, k_ref[...],
        dimension_numbers=(((2,), (2,)), ((0,), (0,))),
        preferred_element_type=jnp.float32,
    )


def _tc_logits(query, key):
    n, p, d = query.shape
    b = MM_BLOCK
    return pl.pallas_call(
        _logits_body,
        grid=(n // b,),
        in_specs=[
            pl.BlockSpec((b, p, d), lambda i: (i, 0, 0)),
            pl.BlockSpec((b, p, d), lambda i: (i, 0, 0)),
        ],
        out_specs=pl.BlockSpec((b, p, p), lambda i: (i, 0, 0)),
        out_shape=jax.ShapeDtypeStruct((n, p, p), jnp.float32),
    )(query, key)


def _sc_topk_body(x_hbm, w_hbm, i_hbm, xbuf0, xbuf1, wbuf, ibuf, sem0, sem1):
    nrows, p = x_hbm.shape
    rows_per_w = nrows // NUM_WORKERS
    nblk = rows_per_w // ROW_BLOCK
    nchunk = p // 16
    wid = lax.axis_index("s") * 2 + lax.axis_index("c")
    wbase = wid * rows_per_w
    lane = lax.broadcasted_iota(jnp.int32, (16,), 0)

    def compute_block(xbuf):
        @plsc.parallel_loop(0, ROW_BLOCK, unroll=2)
        def row_body(r):
            level = []
            for c in range(nchunk):
                v, i = plsc.sort_key_val(
                    xbuf[r, pl.ds(c * 16, 16)], lane + c * 16,
                    descending=(c % 2 == 0))
                level.append((v, i))
            while len(level) > 1:
                nxt = []
                for j in range(0, len(level), 2):
                    av, ai = level[j]
                    bv, bi = level[j + 1]
                    ge = av >= bv
                    nv = jnp.where(ge, av, bv)
                    ni = jnp.where(ge, ai, bi)
                    desc = True if len(level) == 2 else (j // 2) % 2 == 0
                    nxt.append(plsc.sort_key_val(nv, ni, descending=desc))
                level = nxt
            ov, oi = level[0]
            m = lax.reduce_max(ov, axes=(0,))
            e = jnp.exp(ov - m)
            s = lax.reduce_sum(e, axes=(0,))
            wbuf[r] = e / s
            ibuf[r] = oi

    pltpu.async_copy(x_hbm.at[pl.ds(wbase, ROW_BLOCK)], xbuf0, sem0)
    pltpu.async_copy(x_hbm.at[pl.ds(wbase + ROW_BLOCK, ROW_BLOCK)], xbuf1, sem1)

    def pair_body(b2, _):
        for ph, (xb, sem) in enumerate(((xbuf0, sem0), (xbuf1, sem1))):
            b = 2 * b2 + ph
            base = wbase + b * ROW_BLOCK
            pltpu.make_async_copy(
                x_hbm.at[pl.ds(base, ROW_BLOCK)], xb, sem).wait()
            compute_block(xb)

            @pl.when(b + 2 < nblk)
            def _():
                pltpu.async_copy(
                    x_hbm.at[pl.ds(base + 2 * ROW_BLOCK, ROW_BLOCK)], xb, sem)

            pltpu.sync_copy(wbuf, w_hbm.at[pl.ds(base, ROW_BLOCK)])
            pltpu.sync_copy(ibuf, i_hbm.at[pl.ds(base, ROW_BLOCK)])
        return 0

    lax.fori_loop(0, nblk // 2, pair_body, 0)


def _sc_topk(x, n):
    nrows, p = x.shape
    mesh = plsc.VectorSubcoreMesh(
        core_axis_name="c", subcore_axis_name="s",
        num_cores=2, num_subcores=16)
    f = functools.partial(
        pl.kernel,
        mesh=mesh,
        out_type=[
            jax.ShapeDtypeStruct((nrows, TOPK), jnp.float32),
            jax.ShapeDtypeStruct((nrows, TOPK), jnp.int32),
        ],
        scratch_types=[
            pltpu.VMEM((ROW_BLOCK, p), jnp.float32),
            pltpu.VMEM((ROW_BLOCK, p), jnp.float32),
            pltpu.VMEM((ROW_BLOCK, TOPK), jnp.float32),
            pltpu.VMEM((ROW_BLOCK, TOPK), jnp.int32),
            pltpu.SemaphoreType.DMA,
            pltpu.SemaphoreType.DMA,
        ],
        compiler_params=pltpu.CompilerParams(needs_layout_passes=False),
    )(_sc_topk_body)
    return f(x)


@jax.jit
def kernel(query, key):
    n, p, d = query.shape
    x = _tc_logits(query, key)
    w, i = _sc_topk(x.reshape(n * p, p), n)
    return w.reshape(n, p, TOPK), i.reshape(n, p, TOPK)


# speedup vs baseline: 2.4880x; 1.4556x over previous
import functools

import jax
import jax.numpy as jnp
from jax import lax
from jax.experimental import pallas as pl
from jax.experimental.pallas import tpu as pltpu
from jax.experimental.pallas import tpu_sc as plsc

QK_DIM = 32
TOPK = 16
SCALE = QK_DIM ** (-0.5)
MM_BLOCK = 16
ROW_BLOCK = 128
NUM_WORKERS = 32


def _logits_body(q_ref, k_ref, x_ref):
    x_ref[...] = lax.dot_general(
        q_ref[...] * SCALE, k_ref[...],
        dimension_numbers=(((1,), (1,)), ((0,), (0,))),
        preferred_element_type=jnp.float32,
    )


def _tc_logits(qt, kt):
    n, d, p = qt.shape
    b = MM_BLOCK
    return pl.pallas_call(
        _logits_body,
        grid=(n // b,),
        in_specs=[
            pl.BlockSpec((b, d, p), lambda i: (i, 0, 0)),
            pl.BlockSpec((b, d, p), lambda i: (i, 0, 0)),
        ],
        out_specs=pl.BlockSpec((b, p, p), lambda i: (i, 0, 0)),
        out_shape=jax.ShapeDtypeStruct((n, p, p), jnp.float32),
    )(qt, kt)


def _sc_topk_body(x_hbm, w_hbm, i_hbm, xbuf0, xbuf1, wbuf, ibuf, sem0, sem1):
    nrows, p = x_hbm.shape
    rows_per_w = nrows // NUM_WORKERS
    nblk = rows_per_w // ROW_BLOCK
    nchunk = p // 16
    wid = lax.axis_index("s") * 2 + lax.axis_index("c")
    wbase = wid * rows_per_w
    lane = lax.broadcasted_iota(jnp.int32, (16,), 0)

    def compute_block(xbuf):
        @plsc.parallel_loop(0, ROW_BLOCK, unroll=2)
        def row_body(r):
            level = []
            for c in range(nchunk):
                v, i = plsc.sort_key_val(
                    xbuf[r, pl.ds(c * 16, 16)], lane + c * 16,
                    descending=(c % 2 == 0))
                level.append((v, i))
            while len(level) > 1:
                nxt = []
                for j in range(0, len(level), 2):
                    av, ai = level[j]
                    bv, bi = level[j + 1]
                    ge = av >= bv
                    nv = jnp.where(ge, av, bv)
                    ni = jnp.where(ge, ai, bi)
                    desc = True if len(level) == 2 else (j // 2) % 2 == 0
                    nxt.append(plsc.sort_key_val(nv, ni, descending=desc))
                level = nxt
            ov, oi = level[0]
            m = lax.reduce_max(ov, axes=(0,))
            e = jnp.exp(ov - m)
            s = lax.reduce_sum(e, axes=(0,))
            wbuf[r] = e / s
            ibuf[r] = oi

    pltpu.async_copy(x_hbm.at[pl.ds(wbase, ROW_BLOCK)], xbuf0, sem0)
    pltpu.async_copy(x_hbm.at[pl.ds(wbase + ROW_BLOCK, ROW_BLOCK)], xbuf1, sem1)

    def pair_body(b2, _):
        for ph, (xb, sem) in enumerate(((xbuf0, sem0), (xbuf1, sem1))):
            b = 2 * b2 + ph
            base = wbase + b * ROW_BLOCK
            pltpu.make_async_copy(
                x_hbm.at[pl.ds(base, ROW_BLOCK)], xb, sem).wait()
            compute_block(xb)

            @pl.when(b + 2 < nblk)
            def _():
                pltpu.async_copy(
                    x_hbm.at[pl.ds(base + 2 * ROW_BLOCK, ROW_BLOCK)], xb, sem)

            pltpu.sync_copy(wbuf, w_hbm.at[pl.ds(base, ROW_BLOCK)])
            pltpu.sync_copy(ibuf, i_hbm.at[pl.ds(base, ROW_BLOCK)])
        return 0

    lax.fori_loop(0, nblk // 2, pair_body, 0)


def _sc_topk(x, n):
    nrows, p = x.shape
    mesh = plsc.VectorSubcoreMesh(
        core_axis_name="c", subcore_axis_name="s",
        num_cores=2, num_subcores=16)
    f = functools.partial(
        pl.kernel,
        mesh=mesh,
        out_type=[
            jax.ShapeDtypeStruct((nrows, TOPK), jnp.float32),
            jax.ShapeDtypeStruct((nrows, TOPK), jnp.int32),
        ],
        scratch_types=[
            pltpu.VMEM((ROW_BLOCK, p), jnp.float32),
            pltpu.VMEM((ROW_BLOCK, p), jnp.float32),
            pltpu.VMEM((ROW_BLOCK, TOPK), jnp.float32),
            pltpu.VMEM((ROW_BLOCK, TOPK), jnp.int32),
            pltpu.SemaphoreType.DMA,
            pltpu.SemaphoreType.DMA,
        ],
        compiler_params=pltpu.CompilerParams(needs_layout_passes=False),
    )(_sc_topk_body)
    return f(x)


@jax.jit
def kernel(query, key):
    n, p, d = query.shape
    x = _tc_logits(jnp.transpose(query, (0, 2, 1)),
                   jnp.transpose(key, (0, 2, 1)))
    w, i = _sc_topk(x.reshape(n * p, p), n)
    return w.reshape(n, p, TOPK), i.reshape(n, p, TOPK)
